# baseline (speedup 1.0000x reference)
import functools

import jax
import jax.numpy as jnp
from jax import lax
from jax.experimental import pallas as pl
from jax.experimental.pallas import tpu as pltpu
from jax.experimental.pallas import tpu_sc as plsc

_NCB = 4
_K = 8192
_D = 16
_N = 8192
_RT = 256
_NRT = _N // _RT

_NW = 32
_RW = (_N * _NCB) // _NW


def _tree_norm(x2):
    t = x2
    for s in (8, 4, 2, 1):
        t = t[:, 0:s] + t[:, s:2 * s]
    return t


def _split_hilo(v):
    hi = v.astype(jnp.bfloat16).astype(jnp.float32)
    lo = (v - hi).astype(jnp.bfloat16).astype(jnp.float32)
    return hi, lo


def _argmin_body(x_ref, e_ref, ids_ref):
    x = x_ref[...]
    cols = []
    for c in range(_NCB):
        xc = x[:, c * _D:(c + 1) * _D]
        ec = e_ref[c]
        a = _tree_norm(xc * xc)
        b = _tree_norm(ec * ec)[:, 0].reshape(1, _K)
        xh, xl = _split_hilo(xc)
        eh, el = _split_hilo(ec)
        x4 = jnp.concatenate([xh, xh, xl, xl], axis=1)
        e4 = jnp.concatenate([eh, el, eh, el], axis=1)
        mm = lax.dot_general(
            x4, e4, (((1,), (1,)), ((), ())),
            preferred_element_type=jnp.float32)
        d = (a + b) - 2.0 * mm
        m = jnp.min(d, axis=1, keepdims=True)
        iota = lax.broadcasted_iota(jnp.int32, (_RT, _K), 1)
        cand = jnp.where(d == m, iota, jnp.int32(_K))
        cols.append(jnp.min(cand, axis=1, keepdims=True))
    ids_ref[0] = jnp.concatenate(cols, axis=1)


def _run_argmin(lat2, codebooks):
    return pl.pallas_call(
        _argmin_body,
        grid=(_NRT,),
        in_specs=[
            pl.BlockSpec((_RT, _NCB * _D), lambda r: (r, 0)),
            pl.BlockSpec((_NCB, _K, _D), lambda r: (0, 0, 0)),
        ],
        out_specs=pl.BlockSpec((1, _RT, _NCB), lambda r: (r, 0, 0)),
        out_shape=jax.ShapeDtypeStruct((_NRT, _RT, _NCB), jnp.int32),
    )(lat2, codebooks)


def _sc_body(ids_hbm, cb_hbm, w_hbm, quant_hbm, hist_hbm,
             idx_v, gidx_v, rows_v, w_v, hist_v, sem):
    info = plsc.get_sparse_core_info()
    nc = info.num_cores
    wid = lax.axis_index("s") * nc + lax.axis_index("c")
    base = wid * _RW

    pltpu.sync_copy(ids_hbm.at[pl.ds(base, _RW)], idx_v)
    pltpu.sync_copy(w_hbm.at[pl.ds(base, _RW)], w_v)

    def _zero(i, carry):
        hist_v[pl.ds(i * 16, 16)] = jnp.zeros((16,), jnp.float32)
        return carry
    lax.fori_loop(0, (_NCB * _K) // 16, _zero, 0)

    lane = lax.iota(jnp.int32, 16)
    offs = (lane % _NCB) * _K
    for t in range(_RW // 16):
        v = idx_v[pl.ds(t * 16, 16)]
        g = v + offs
        gidx_v[t // 8, pl.ds((t % 8) * 16, 16)] = g
        wv = w_v[pl.ds(t * 16, 16)]
        plsc.addupdate_scatter(hist_v, [g], wv)

    copies = []
    for k in range(_RW // 128):
        copies.append(pltpu.async_copy(
            cb_hbm.at[gidx_v.at[k]], rows_v.at[pl.ds(k * 128, 128)], sem))
    for cp in copies:
        cp.wait()

    pltpu.sync_copy(rows_v, quant_hbm.at[pl.ds(base, _RW)])
    pltpu.sync_copy(hist_v, hist_hbm.at[wid])


def _run_sc(ids_flat, cb_flat, w4):
    mesh = plsc.VectorSubcoreMesh(core_axis_name="c", subcore_axis_name="s")
    kern = functools.partial(
        pl.kernel,
        mesh=mesh,
        compiler_params=pltpu.CompilerParams(
            needs_layout_passes=False, use_tc_tiling_on_sc=False),
        out_type=[
            jax.ShapeDtypeStruct((_N * _NCB, _D), jnp.float32),
            jax.ShapeDtypeStruct((_NW, _NCB * _K), jnp.float32),
        ],
        scratch_types=[
            pltpu.VMEM((_RW,), jnp.int32),
            pltpu.VMEM((_RW // 128, 128), jnp.int32),
            pltpu.VMEM((_RW, _D), jnp.float32),
            pltpu.VMEM((_RW,), jnp.float32),
            pltpu.VMEM((_NCB * _K,), jnp.float32),
            pltpu.SemaphoreType.DMA,
        ],
    )(_sc_body)
    return kern(ids_flat, cb_flat, w4)


def _finish_body(l_ref, q_ref, st_ref, loss_ref):
    l = l_ref[...]
    q = q_ref[...]
    st_ref[...] = l + (q - l)
    d = l - q
    loss_ref[0, 0] = jnp.sum(d * d)


def _run_finish(lat2, qflat):
    return pl.pallas_call(
        _finish_body,
        in_specs=[
            pl.BlockSpec((_N, _NCB * _D), lambda: (0, 0)),
            pl.BlockSpec((_N, _NCB * _D), lambda: (0, 0)),
        ],
        out_specs=[
            pl.BlockSpec((_N, _NCB * _D), lambda: (0, 0)),
            pl.BlockSpec(memory_space=pltpu.SMEM),
        ],
        out_shape=[
            jax.ShapeDtypeStruct((_N, _NCB * _D), jnp.float32),
            jax.ShapeDtypeStruct((1, 1), jnp.float32),
        ],
    )(lat2, qflat)


def _perp_body(hp_ref, w_ref, perp_ref):
    hist = jnp.sum(hp_ref[...], axis=0, keepdims=True)
    w = w_ref[...]
    denom = jnp.maximum(jnp.sum(w), 1.0)
    p = hist / denom
    s = jnp.sum(p * jnp.log(p + 1e-08))
    perp_ref[pl.program_id(0), 0] = jnp.exp(-s)


def _run_perp(histp, w2d):
    return pl.pallas_call(
        _perp_body,
        grid=(_NCB,),
        in_specs=[
            pl.BlockSpec((_NW, _K), lambda c: (0, c)),
            pl.BlockSpec((1, _N), lambda c: (0, 0)),
        ],
        out_specs=pl.BlockSpec((_NCB, 1), lambda c: (0, 0),
                               memory_space=pltpu.SMEM),
        out_shape=jax.ShapeDtypeStruct((_NCB, 1), jnp.float32),
    )(histp, w2d)


def kernel(latents, patch_mask, codebooks):
    bsz, npatch, ldim = latents.shape
    lat2 = latents.reshape(_N, ldim)
    w_flat = patch_mask.reshape(-1)
    w4 = jnp.repeat(w_flat, _NCB)
    cb_flat = codebooks.reshape(_NCB * _K, _D)

    ids3 = _run_argmin(lat2, codebooks)
    ids_pairs = ids3.reshape(_N * _NCB)

    quant2, histp = _run_sc(ids_pairs, cb_flat, w4)
    qflat = quant2.reshape(_N, _NCB * _D)

    st2, loss = _run_finish(lat2, qflat)
    perp4 = _run_perp(histp, w_flat.reshape(1, _N))

    symbol_ids = ids3.reshape(bsz, npatch, _NCB)
    quantized = qflat.reshape(bsz, npatch, ldim)
    st_quantized = st2.reshape(bsz, npatch, ldim)
    total = loss[0, 0] / jnp.float32(_NCB * _N * _D)
    commitment_loss = total * 0.25
    codebook_loss = total * 1.0
    perplexity = jnp.mean(perp4[:, 0])
    return (symbol_ids, quantized, st_quantized,
            commitment_loss, codebook_loss, perplexity)
